# baseline (speedup 1.0000x reference)
import functools

import jax
import jax.numpy as jnp
from jax import lax
from jax.experimental import pallas as pl
from jax.experimental.pallas import tpu as pltpu
from jax.experimental.pallas import tpu_sc as plsc

_T = 0.5
_EPS = 1e-9
_W = 16
_L = 16


def _sc_nms(np_):
    nb = np_ // _W
    nc = nb // _L
    mesh = plsc.VectorSubcoreMesh(core_axis_name="c", subcore_axis_name="s")

    def body(b0h, b1h, b2h, b3h, ssh, ksh, kph,
             b0v, b1v, b2v, b3v, x1, y1, x2, y2, ar,
             keepb, kbb, ssb, osc, keep_sh):
        cid = lax.axis_index("c")
        w = lax.axis_index("s")
        active = cid == 0
        base = w * nb
        lane = lax.broadcasted_iota(jnp.int32, (_L,), 0)

        def extract(ref, g):
            return ref[pl.ds(g, _L)][0]

        @pl.when(active)
        def _stage():
            pltpu.sync_copy(b0h, b0v.at[pl.ds(0, np_)])
            pltpu.sync_copy(b1h, b1v.at[pl.ds(0, np_)])
            pltpu.sync_copy(b2h, b2v.at[pl.ds(0, np_)])
            pltpu.sync_copy(b3h, b3v.at[pl.ds(0, np_)])
            pltpu.sync_copy(ssh.at[pl.ds(base, nb)], ssb)

            def norm(i, _):
                o = i * _L
                c0 = b0v[pl.ds(o, _L)]
                c1 = b1v[pl.ds(o, _L)]
                c2 = b2v[pl.ds(o, _L)]
                c3 = b3v[pl.ds(o, _L)]
                vx1 = jnp.minimum(c0, c2)
                vy1 = jnp.minimum(c1, c3)
                vx2 = jnp.maximum(c0, c2)
                vy2 = jnp.maximum(c1, c3)
                x1[pl.ds(o, _L)] = vx1
                y1[pl.ds(o, _L)] = vy1
                x2[pl.ds(o, _L)] = vx2
                y2[pl.ds(o, _L)] = vy2
                ar[pl.ds(o, _L)] = (vx2 - vx1) * (vy2 - vy1)
                return 0

            lax.fori_loop(0, np_ // _L, norm, 0)

            def initk(i, _):
                keepb[pl.ds(i * _L, _L)] = jnp.ones((_L,), jnp.float32)
                return 0

            lax.fori_loop(0, nc, initk, 0)

        def suppress_by(g, c_lo, unroll=1):
            sx1 = extract(x1, g)
            sy1 = extract(y1, g)
            sx2 = extract(x2, g)
            sy2 = extract(y2, g)
            sar = extract(ar, g)

            def sub(c, _):
                o = c * _L
                tx1 = x1[pl.ds(base + o, _L)]
                ty1 = y1[pl.ds(base + o, _L)]
                tx2 = x2[pl.ds(base + o, _L)]
                ty2 = y2[pl.ds(base + o, _L)]
                tar = ar[pl.ds(base + o, _L)]
                xx1 = jnp.maximum(tx1, sx1)
                yy1 = jnp.maximum(ty1, sy1)
                xx2 = jnp.minimum(tx2, sx2)
                yy2 = jnp.minimum(ty2, sy2)
                iw = jnp.maximum(xx2 - xx1, 0.0)
                ih = jnp.maximum(yy2 - yy1, 0.0)
                inter = iw * ih
                union = (sar + tar) - inter
                m = (inter / (union + _EPS)) > _T
                m = m & ((base + o + lane) > g)
                kc = keepb[pl.ds(o, _L)]
                keepb[pl.ds(o, _L)] = jnp.where(m, 0.0, kc)
                return 0

            lax.fori_loop(c_lo, nc, sub, 0, unroll=unroll)

        def biter(b, _):
            @pl.when(active & (w == b))
            def _intra():
                def step(k, _):
                    kk = extract(keepb, k)

                    @pl.when(kk > 0.0)
                    def _s():
                        suppress_by(base + k, k // _L)

                    return 0

                lax.fori_loop(0, nb, step, 0)
                pltpu.sync_copy(keepb.at[pl.ds(0, nb)], keep_sh.at[pl.ds(w * nb, nb)])

            plsc.subcore_barrier()

            @pl.when(active & (w > b))
            def _cross():
                pltpu.sync_copy(keep_sh.at[pl.ds(b * nb, nb)], kbb.at[pl.ds(0, nb)])

                def stepc(c, _):
                    kchunk = kbb[pl.ds(c * _L, _L)]
                    for i in range(_L):
                        kk = kchunk[i]

                        @pl.when(kk > 0.0)
                        def _s(c=c, i=i, kk=kk):
                            suppress_by(b * nb + c * _L + i, 0, unroll=2)

                    return 0

                lax.fori_loop(0, nc, stepc, 0)

            return 0

        lax.fori_loop(0, _W, biter, 0)

        @pl.when(active)
        def _out():
            def fin(c, _):
                o = c * _L
                osc[pl.ds(o, _L)] = ssb[pl.ds(o, _L)] * keepb[pl.ds(o, _L)]
                return 0

            lax.fori_loop(0, nc, fin, 0)
            pltpu.sync_copy(osc, ksh.at[pl.ds(base, nb)])
            pltpu.sync_copy(keepb.at[pl.ds(0, nb)], kph.at[pl.ds(base, nb)])

    f = functools.partial(
        pl.kernel,
        out_type=[
            jax.ShapeDtypeStruct((np_,), jnp.float32),
            jax.ShapeDtypeStruct((np_,), jnp.float32),
        ],
        mesh=mesh,
        scratch_types=[
            pltpu.VMEM((np_ + _L,), jnp.float32),
            pltpu.VMEM((np_ + _L,), jnp.float32),
            pltpu.VMEM((np_ + _L,), jnp.float32),
            pltpu.VMEM((np_ + _L,), jnp.float32),
            pltpu.VMEM((np_ + _L,), jnp.float32),
            pltpu.VMEM((np_ + _L,), jnp.float32),
            pltpu.VMEM((np_ + _L,), jnp.float32),
            pltpu.VMEM((np_ + _L,), jnp.float32),
            pltpu.VMEM((np_ + _L,), jnp.float32),
            pltpu.VMEM((nb + _L,), jnp.float32),
            pltpu.VMEM((nb + _L,), jnp.float32),
            pltpu.VMEM((nb,), jnp.float32),
            pltpu.VMEM((nb,), jnp.float32),
            pltpu.HBM((_W * nb,), jnp.float32),
        ],
    )(body)
    return f


def kernel(boxes, scores):
    n = scores.shape[0]
    order = jnp.argsort(-scores)
    sb = boxes[order]
    ss = scores[order]
    np_ = ((n + 255) // 256) * 256
    pad = np_ - n
    sb_p = jnp.pad(sb, ((0, pad), (0, 0)))
    ss_p = jnp.pad(ss, ((0, pad),))

    ks_p, kp_p = _sc_nms(np_)(
        sb_p[:, 0], sb_p[:, 1], sb_p[:, 2], sb_p[:, 3], ss_p)

    keep = kp_p[:n] > 0.5
    return ks_p[:n], order, keep


# speedup vs baseline: 1.0296x; 1.0296x over previous
import functools

import jax
import jax.numpy as jnp
from jax import lax
from jax.experimental import pallas as pl
from jax.experimental.pallas import tpu as pltpu
from jax.experimental.pallas import tpu_sc as plsc

_T = 0.5
_EPS = 1e-9
_W = 16
_L = 16


def _sc_nms(np_):
    nb = np_ // _W
    nc = nb // _L
    mesh = plsc.VectorSubcoreMesh(core_axis_name="c", subcore_axis_name="s")

    def body(b0h, b1h, b2h, b3h, ssh, ksh, kph,
             b0v, b1v, b2v, b3v, x1, y1, x2, y2, ar,
             keepb, kbb, ssb, osc, keep_sh):
        cid = lax.axis_index("c")
        w = lax.axis_index("s")
        active = cid == 0
        base = w * nb
        lane = lax.broadcasted_iota(jnp.int32, (_L,), 0)

        def extract(ref, g):
            return ref[pl.ds(g, _L)][0]

        @pl.when(active)
        def _stage():
            pltpu.sync_copy(b0h, b0v.at[pl.ds(0, np_)])
            pltpu.sync_copy(b1h, b1v.at[pl.ds(0, np_)])
            pltpu.sync_copy(b2h, b2v.at[pl.ds(0, np_)])
            pltpu.sync_copy(b3h, b3v.at[pl.ds(0, np_)])
            pltpu.sync_copy(ssh.at[pl.ds(base, nb)], ssb)

            def norm(i, _):
                o = i * _L
                c0 = b0v[pl.ds(o, _L)]
                c1 = b1v[pl.ds(o, _L)]
                c2 = b2v[pl.ds(o, _L)]
                c3 = b3v[pl.ds(o, _L)]
                vx1 = jnp.minimum(c0, c2)
                vy1 = jnp.minimum(c1, c3)
                vx2 = jnp.maximum(c0, c2)
                vy2 = jnp.maximum(c1, c3)
                x1[pl.ds(o, _L)] = vx1
                y1[pl.ds(o, _L)] = vy1
                x2[pl.ds(o, _L)] = vx2
                y2[pl.ds(o, _L)] = vy2
                ar[pl.ds(o, _L)] = (vx2 - vx1) * (vy2 - vy1)
                return 0

            lax.fori_loop(0, np_ // _L, norm, 0)

            def initk(i, _):
                keepb[pl.ds(i * _L, _L)] = jnp.ones((_L,), jnp.float32)
                return 0

            lax.fori_loop(0, nc, initk, 0)

        def suppress_by(g, c_lo):
            sx1 = extract(x1, g)
            sy1 = extract(y1, g)
            sx2 = extract(x2, g)
            sy2 = extract(y2, g)
            sar = extract(ar, g)

            def sub(c, _):
                o = c * _L
                tx1 = x1[pl.ds(base + o, _L)]
                ty1 = y1[pl.ds(base + o, _L)]
                tx2 = x2[pl.ds(base + o, _L)]
                ty2 = y2[pl.ds(base + o, _L)]
                tar = ar[pl.ds(base + o, _L)]
                xx1 = jnp.maximum(tx1, sx1)
                yy1 = jnp.maximum(ty1, sy1)
                xx2 = jnp.minimum(tx2, sx2)
                yy2 = jnp.minimum(ty2, sy2)
                iw = jnp.maximum(xx2 - xx1, 0.0)
                ih = jnp.maximum(yy2 - yy1, 0.0)
                inter = iw * ih
                union = (sar + tar) - inter
                m = (inter / (union + _EPS)) > _T
                m = m & ((base + o + lane) > g)
                kc = keepb[pl.ds(o, _L)]
                keepb[pl.ds(o, _L)] = jnp.where(m, 0.0, kc)
                return 0

            lax.fori_loop(c_lo, nc, sub, 0)

        def biter(b, _):
            @pl.when(active & (w == b))
            def _intra():
                def step(k, _):
                    kk = extract(keepb, k)

                    @pl.when(kk > 0.0)
                    def _s():
                        suppress_by(base + k, k // _L)

                    return 0

                lax.fori_loop(0, nb, step, 0)
                pltpu.sync_copy(keepb.at[pl.ds(0, nb)], keep_sh.at[pl.ds(w * nb, nb)])

            plsc.subcore_barrier()

            @pl.when(active & (w > b))
            def _cross():
                pltpu.sync_copy(keep_sh.at[pl.ds(b * nb, nb)], kbb.at[pl.ds(0, nb)])

                def step(k, _):
                    kk = extract(kbb, k)

                    @pl.when(kk > 0.0)
                    def _s():
                        suppress_by(b * nb + k, 0)

                    return 0

                lax.fori_loop(0, nb, step, 0)

            return 0

        lax.fori_loop(0, _W, biter, 0)

        @pl.when(active)
        def _out():
            def fin(c, _):
                o = c * _L
                osc[pl.ds(o, _L)] = ssb[pl.ds(o, _L)] * keepb[pl.ds(o, _L)]
                return 0

            lax.fori_loop(0, nc, fin, 0)
            pltpu.sync_copy(osc, ksh.at[pl.ds(base, nb)])
            pltpu.sync_copy(keepb.at[pl.ds(0, nb)], kph.at[pl.ds(base, nb)])

    f = functools.partial(
        pl.kernel,
        out_type=[
            jax.ShapeDtypeStruct((np_,), jnp.float32),
            jax.ShapeDtypeStruct((np_,), jnp.float32),
        ],
        mesh=mesh,
        scratch_types=[
            pltpu.VMEM((np_ + _L,), jnp.float32),
            pltpu.VMEM((np_ + _L,), jnp.float32),
            pltpu.VMEM((np_ + _L,), jnp.float32),
            pltpu.VMEM((np_ + _L,), jnp.float32),
            pltpu.VMEM((np_ + _L,), jnp.float32),
            pltpu.VMEM((np_ + _L,), jnp.float32),
            pltpu.VMEM((np_ + _L,), jnp.float32),
            pltpu.VMEM((np_ + _L,), jnp.float32),
            pltpu.VMEM((np_ + _L,), jnp.float32),
            pltpu.VMEM((nb + _L,), jnp.float32),
            pltpu.VMEM((nb + _L,), jnp.float32),
            pltpu.VMEM((nb,), jnp.float32),
            pltpu.VMEM((nb,), jnp.float32),
            pltpu.HBM((_W * nb,), jnp.float32),
        ],
    )(body)
    return f


def kernel(boxes, scores):
    n = scores.shape[0]
    order = jnp.argsort(-scores)
    sb = boxes[order]
    ss = scores[order]
    np_ = ((n + 255) // 256) * 256
    pad = np_ - n
    sb_p = jnp.pad(sb, ((0, pad), (0, 0)))
    ss_p = jnp.pad(ss, ((0, pad),))

    ks_p, kp_p = _sc_nms(np_)(
        sb_p[:, 0], sb_p[:, 1], sb_p[:, 2], sb_p[:, 3], ss_p)

    keep = kp_p[:n] > 0.5
    return ks_p[:n], order, keep
